# baseline (speedup 1.0000x reference)
import functools

import jax
import jax.numpy as jnp
from jax import lax
from jax.experimental import pallas as pl
from jax.experimental.pallas import tpu as pltpu
from jax.experimental.pallas import tpu_sc as plsc

N_NODES = 10000
IN_DIM = 128
OUT_DIM = 128
N_EDGES = 320000

NC = 2
NS = 16
NW = NC * NS
CHUNK = 128
CPW = 80
E_PAD = NW * CPW * CHUNK
N_PAD = 10240
ROWS_PER_TILE = N_PAD // NS
NBUF = 2
PHASE = CPW // 2


def _mm_body(h_ref, w_ref, y_ref):
    y = lax.dot_general(h_ref[...], w_ref[...],
                        dimension_numbers=(((1,), (1,)), ((), ())),
                        preferred_element_type=jnp.float32)
    y_ref[...] = jnp.maximum(y, 0.0)


def _relu_mm(h, w):
    m = 1000
    return pl.pallas_call(
        _mm_body,
        grid=(N_NODES // m,),
        in_specs=[pl.BlockSpec((m, IN_DIM), lambda i: (i, 0)),
                  pl.BlockSpec((OUT_DIM, IN_DIM), lambda i: (0, 0))],
        out_specs=pl.BlockSpec((m, OUT_DIM), lambda i: (i, 0)),
        out_shape=jax.ShapeDtypeStruct((N_NODES, OUT_DIM), jnp.float32),
    )(h, w)


_sc_mesh = plsc.VectorSubcoreMesh(core_axis_name="c", subcore_axis_name="s")


@functools.partial(
    pl.kernel,
    mesh=_sc_mesh,
    out_type=jax.ShapeDtypeStruct((NC, N_PAD, OUT_DIM), jnp.float32),
    scratch_types=[
        pltpu.VMEM((PHASE, CHUNK), jnp.int32),
        pltpu.VMEM((PHASE, CHUNK), jnp.int32),
        pltpu.VMEM((NBUF, CHUNK, OUT_DIM), jnp.float32),
        pltpu.VMEM_SHARED((N_PAD, OUT_DIM), jnp.float32),
        pltpu.SemaphoreType.DMA((NBUF,)),
        pltpu.SemaphoreType.DMA((NBUF,)),
        pltpu.SemaphoreType.DMA,
    ],
)
def _sc_scatter(y_hbm, src_hbm, dst_hbm, zeros_hbm, out_hbm,
                sidx, didx, rows, acc, gsem, ssem, zsem):
    c = lax.axis_index("c")
    s = lax.axis_index("s")
    w = c * NS + s

    zero_cp = pltpu.async_copy(
        zeros_hbm, acc.at[pl.ds(s * ROWS_PER_TILE, ROWS_PER_TILE)], zsem)

    def gissue(i, b):
        h = CHUNK // 2
        pltpu.async_copy(y_hbm.at[sidx.at[i, pl.ds(0, h)]],
                         rows.at[b, pl.ds(0, h)], gsem.at[b])
        pltpu.async_copy(y_hbm.at[sidx.at[i, pl.ds(h, h)]],
                         rows.at[b, pl.ds(h, h)], gsem.at[b])

    def gwait(b):
        pltpu.make_async_copy(y_hbm.at[pl.ds(0, CHUNK)], rows.at[b],
                              gsem.at[b]).wait()

    def swait(b):
        pltpu.make_async_copy(y_hbm.at[pl.ds(0, CHUNK)], rows.at[b],
                              ssem.at[b]).wait()

    for p in range(CPW // PHASE):
        base_row = w * CPW + p * PHASE
        pltpu.sync_copy(src_hbm.at[pl.ds(base_row, PHASE)], sidx)
        pltpu.sync_copy(dst_hbm.at[pl.ds(base_row, PHASE)], didx)

        gissue(0, 0)

        if p == 0:
            zero_cp.wait()
            plsc.subcore_barrier()

        def body(j, carry):
            for b in range(NBUF):
                i = j * NBUF + b
                ob = 1 - b
                gwait(b)
                pltpu.async_copy(rows.at[b], acc.at[didx.at[i]],
                                 ssem.at[b], add=True)

                @pl.when(i >= 1)
                def _():
                    swait(ob)

                @pl.when(i + 1 < PHASE)
                def _():
                    gissue(i + 1, ob)
            return carry

        lax.fori_loop(0, PHASE // NBUF, body, 0)
        swait((PHASE - 1) % NBUF)
    plsc.subcore_barrier()

    pltpu.sync_copy(acc.at[pl.ds(s * ROWS_PER_TILE, ROWS_PER_TILE)],
                    out_hbm.at[c, pl.ds(s * ROWS_PER_TILE, ROWS_PER_TILE)])


def _combine_body(p_ref, o_ref):
    o_ref[...] = p_ref[0] + p_ref[1]


def _combine(p):
    m = 1000
    return pl.pallas_call(
        _combine_body,
        grid=(N_NODES // m,),
        in_specs=[pl.BlockSpec((NC, m, OUT_DIM), lambda i: (0, i, 0))],
        out_specs=pl.BlockSpec((m, OUT_DIM), lambda i: (i, 0)),
        out_shape=jax.ShapeDtypeStruct((N_NODES, OUT_DIM), jnp.float32),
    )(p)


def kernel(h, edge_index, W):
    src = edge_index[0].astype(jnp.int32)
    dst = edge_index[1].astype(jnp.int32)
    pad = E_PAD - N_EDGES
    pad_idx = jnp.arange(pad, dtype=jnp.int32)
    src_p = jnp.concatenate([src, pad_idx % N_NODES]).reshape(E_PAD // CHUNK,
                                                              CHUNK)
    dst_p = jnp.concatenate(
        [dst, N_NODES + pad_idx % (N_PAD - N_NODES)]).reshape(E_PAD // CHUNK,
                                                              CHUNK)
    y = _relu_mm(h, W)
    zeros = jnp.zeros((ROWS_PER_TILE, OUT_DIM), jnp.float32)
    partial = _sc_scatter(y, src_p, dst_p, zeros)
    return _combine(partial)


# speedup vs baseline: 1.0010x; 1.0010x over previous
import functools

import jax
import jax.numpy as jnp
from jax import lax
from jax.experimental import pallas as pl
from jax.experimental.pallas import tpu as pltpu
from jax.experimental.pallas import tpu_sc as plsc

N_NODES = 10000
IN_DIM = 128
OUT_DIM = 128
N_EDGES = 320000

NC = 2
NS = 16
NW = NC * NS
CHUNK = 128
CPW = 80
E_PAD = NW * CPW * CHUNK
N_PAD = 10240
ROWS_PER_TILE = N_PAD // NS
NBUF = 2
PHASE = CPW // 2


def _mm_body(h_ref, w_ref, y_ref):
    y = lax.dot_general(h_ref[...], w_ref[...],
                        dimension_numbers=(((1,), (1,)), ((), ())),
                        preferred_element_type=jnp.float32)
    y_ref[...] = jnp.maximum(y, 0.0)


def _relu_mm(h, w):
    m = 1000
    return pl.pallas_call(
        _mm_body,
        grid=(N_NODES // m,),
        in_specs=[pl.BlockSpec((m, IN_DIM), lambda i: (i, 0)),
                  pl.BlockSpec((OUT_DIM, IN_DIM), lambda i: (0, 0))],
        out_specs=pl.BlockSpec((m, OUT_DIM), lambda i: (i, 0)),
        out_shape=jax.ShapeDtypeStruct((N_NODES, OUT_DIM), jnp.float32),
    )(h, w)


_sc_mesh = plsc.VectorSubcoreMesh(core_axis_name="c", subcore_axis_name="s")


@functools.partial(
    pl.kernel,
    mesh=_sc_mesh,
    out_type=jax.ShapeDtypeStruct((NC, N_PAD, OUT_DIM), jnp.float32),
    scratch_types=[
        pltpu.VMEM((PHASE, CHUNK), jnp.int32),
        pltpu.VMEM((PHASE, CHUNK), jnp.int32),
        pltpu.VMEM((NBUF, CHUNK, OUT_DIM), jnp.float32),
        pltpu.VMEM_SHARED((N_PAD, OUT_DIM), jnp.float32),
        pltpu.SemaphoreType.DMA((NBUF,)),
        pltpu.SemaphoreType.DMA((NBUF,)),
        pltpu.SemaphoreType.DMA,
    ],
)
def _sc_scatter(y_hbm, src_hbm, dst_hbm, zeros_hbm, out_hbm,
                sidx, didx, rows, acc, gsem, ssem, zsem):
    c = lax.axis_index("c")
    s = lax.axis_index("s")
    w = c * NS + s

    zero_cp = pltpu.async_copy(
        zeros_hbm, acc.at[pl.ds(s * ROWS_PER_TILE, ROWS_PER_TILE)], zsem)

    def gissue(i, b):
        h = CHUNK // 4
        for q in range(4):
            pltpu.async_copy(y_hbm.at[sidx.at[i, pl.ds(q * h, h)]],
                             rows.at[b, pl.ds(q * h, h)], gsem.at[b])

    def gwait(b):
        pltpu.make_async_copy(y_hbm.at[pl.ds(0, CHUNK)], rows.at[b],
                              gsem.at[b]).wait()

    def swait(b):
        pltpu.make_async_copy(y_hbm.at[pl.ds(0, CHUNK)], rows.at[b],
                              ssem.at[b]).wait()

    for p in range(CPW // PHASE):
        base_row = w * CPW + p * PHASE
        pltpu.sync_copy(src_hbm.at[pl.ds(base_row, PHASE)], sidx)
        pltpu.sync_copy(dst_hbm.at[pl.ds(base_row, PHASE)], didx)

        gissue(0, 0)

        if p == 0:
            zero_cp.wait()
            plsc.subcore_barrier()

        def body(j, carry):
            for b in range(NBUF):
                i = j * NBUF + b
                ob = 1 - b
                gwait(b)
                pltpu.async_copy(rows.at[b], acc.at[didx.at[i]],
                                 ssem.at[b], add=True)

                @pl.when(i >= 1)
                def _():
                    swait(ob)

                @pl.when(i + 1 < PHASE)
                def _():
                    gissue(i + 1, ob)
            return carry

        lax.fori_loop(0, PHASE // NBUF, body, 0)
        swait((PHASE - 1) % NBUF)
    plsc.subcore_barrier()

    pltpu.sync_copy(acc.at[pl.ds(s * ROWS_PER_TILE, ROWS_PER_TILE)],
                    out_hbm.at[c, pl.ds(s * ROWS_PER_TILE, ROWS_PER_TILE)])


def _combine_body(p_ref, o_ref):
    o_ref[...] = p_ref[0] + p_ref[1]


def _combine(p):
    m = 1000
    return pl.pallas_call(
        _combine_body,
        grid=(N_NODES // m,),
        in_specs=[pl.BlockSpec((NC, m, OUT_DIM), lambda i: (0, i, 0))],
        out_specs=pl.BlockSpec((m, OUT_DIM), lambda i: (i, 0)),
        out_shape=jax.ShapeDtypeStruct((N_NODES, OUT_DIM), jnp.float32),
    )(p)


def kernel(h, edge_index, W):
    src = edge_index[0].astype(jnp.int32)
    dst = edge_index[1].astype(jnp.int32)
    pad = E_PAD - N_EDGES
    pad_idx = jnp.arange(pad, dtype=jnp.int32)
    src_p = jnp.concatenate([src, pad_idx % N_NODES]).reshape(E_PAD // CHUNK,
                                                              CHUNK)
    dst_p = jnp.concatenate(
        [dst, N_NODES + pad_idx % (N_PAD - N_NODES)]).reshape(E_PAD // CHUNK,
                                                              CHUNK)
    y = _relu_mm(h, W)
    zeros = jnp.zeros((ROWS_PER_TILE, OUT_DIM), jnp.float32)
    partial = _sc_scatter(y, src_p, dst_p, zeros)
    return _combine(partial)


# speedup vs baseline: 1.0956x; 1.0945x over previous
import functools

import jax
import jax.numpy as jnp
from jax import lax
from jax.experimental import pallas as pl
from jax.experimental.pallas import tpu as pltpu
from jax.experimental.pallas import tpu_sc as plsc

N_NODES = 10000
IN_DIM = 128
OUT_DIM = 128
N_EDGES = 320000

NC = 2
NS = 16
NW = NC * NS
CHUNK = 64
CPW = 160
E_PAD = NW * CPW * CHUNK
N_PAD = 10240
ROWS_PER_TILE = N_PAD // NS
NBUF = 4
PHASE = 40


def _mm_body(h_ref, w_ref, y_ref):
    y = lax.dot_general(h_ref[...], w_ref[...],
                        dimension_numbers=(((1,), (1,)), ((), ())),
                        preferred_element_type=jnp.float32)
    y_ref[...] = jnp.maximum(y, 0.0)


def _relu_mm(h, w):
    m = 1000
    return pl.pallas_call(
        _mm_body,
        grid=(N_NODES // m,),
        in_specs=[pl.BlockSpec((m, IN_DIM), lambda i: (i, 0)),
                  pl.BlockSpec((OUT_DIM, IN_DIM), lambda i: (0, 0))],
        out_specs=pl.BlockSpec((m, OUT_DIM), lambda i: (i, 0)),
        out_shape=jax.ShapeDtypeStruct((N_NODES, OUT_DIM), jnp.float32),
    )(h, w)


_sc_mesh = plsc.VectorSubcoreMesh(core_axis_name="c", subcore_axis_name="s")


@functools.partial(
    pl.kernel,
    mesh=_sc_mesh,
    out_type=jax.ShapeDtypeStruct((NC, N_PAD, OUT_DIM), jnp.float32),
    scratch_types=[
        pltpu.VMEM((PHASE, CHUNK), jnp.int32),
        pltpu.VMEM((PHASE, CHUNK), jnp.int32),
        pltpu.VMEM((NBUF, CHUNK, OUT_DIM), jnp.float32),
        pltpu.VMEM_SHARED((N_PAD, OUT_DIM), jnp.float32),
        pltpu.SemaphoreType.DMA((NBUF,)),
        pltpu.SemaphoreType.DMA((NBUF,)),
        pltpu.SemaphoreType.DMA,
    ],
)
def _sc_scatter(y_hbm, src_hbm, dst_hbm, zeros_hbm, out_hbm,
                sidx, didx, rows, acc, gsem, ssem, zsem):
    c = lax.axis_index("c")
    s = lax.axis_index("s")
    w = c * NS + s

    zero_cp = pltpu.async_copy(
        zeros_hbm, acc.at[pl.ds(s * ROWS_PER_TILE, ROWS_PER_TILE)], zsem)

    def gissue(i, b):
        h = CHUNK // 2
        for q in range(2):
            pltpu.async_copy(y_hbm.at[sidx.at[i, pl.ds(q * h, h)]],
                             rows.at[b, pl.ds(q * h, h)], gsem.at[b])

    def gwait(b):
        pltpu.make_async_copy(y_hbm.at[pl.ds(0, CHUNK)], rows.at[b],
                              gsem.at[b]).wait()

    def swait(b):
        pltpu.make_async_copy(y_hbm.at[pl.ds(0, CHUNK)], rows.at[b],
                              ssem.at[b]).wait()

    for p in range(CPW // PHASE):
        base_row = w * CPW + p * PHASE
        pltpu.sync_copy(src_hbm.at[pl.ds(base_row, PHASE)], sidx)
        pltpu.sync_copy(dst_hbm.at[pl.ds(base_row, PHASE)], didx)

        for k in range(NBUF - 1):
            gissue(k, k)

        if p == 0:
            zero_cp.wait()
            plsc.subcore_barrier()

        def body(j, carry):
            for k in range(NBUF):
                i = j * NBUF + k
                pb = (k - 1) % NBUF
                gwait(k)
                pltpu.async_copy(rows.at[k], acc.at[didx.at[i]],
                                 ssem.at[k], add=True)

                @pl.when(i >= 1)
                def _():
                    swait(pb)

                @pl.when(i + NBUF - 1 < PHASE)
                def _():
                    gissue(i + NBUF - 1, pb)
            return carry

        lax.fori_loop(0, PHASE // NBUF, body, 0)
        swait((PHASE - 1) % NBUF)
    plsc.subcore_barrier()

    pltpu.sync_copy(acc.at[pl.ds(s * ROWS_PER_TILE, ROWS_PER_TILE)],
                    out_hbm.at[c, pl.ds(s * ROWS_PER_TILE, ROWS_PER_TILE)])


def _combine_body(p_ref, o_ref):
    o_ref[...] = p_ref[0] + p_ref[1]


def _combine(p):
    m = 1000
    return pl.pallas_call(
        _combine_body,
        grid=(N_NODES // m,),
        in_specs=[pl.BlockSpec((NC, m, OUT_DIM), lambda i: (0, i, 0))],
        out_specs=pl.BlockSpec((m, OUT_DIM), lambda i: (i, 0)),
        out_shape=jax.ShapeDtypeStruct((N_NODES, OUT_DIM), jnp.float32),
    )(p)


def kernel(h, edge_index, W):
    src = edge_index[0].astype(jnp.int32)
    dst = edge_index[1].astype(jnp.int32)
    pad = E_PAD - N_EDGES
    pad_idx = jnp.arange(pad, dtype=jnp.int32)
    src_p = jnp.concatenate([src, pad_idx % N_NODES]).reshape(E_PAD // CHUNK,
                                                              CHUNK)
    dst_p = jnp.concatenate(
        [dst, N_NODES + pad_idx % (N_PAD - N_NODES)]).reshape(E_PAD // CHUNK,
                                                              CHUNK)
    y = _relu_mm(h, W)
    zeros = jnp.zeros((ROWS_PER_TILE, OUT_DIM), jnp.float32)
    partial = _sc_scatter(y, src_p, dst_p, zeros)
    return _combine(partial)
